# baseline (speedup 1.0000x reference)
import functools

import jax
import jax.numpy as jnp
from jax import lax
from jax.experimental import pallas as pl
from jax.experimental.pallas import tpu as pltpu
from jax.experimental.pallas import tpu_sc as plsc

B, E, N, D = 2, 160000, 10000, 128
NS = 16
CHUNK = 160
SUB = 20
SUBS = CHUNK // SUB
NCHUNK = E // CHUNK
KFULL = NCHUNK // NS
NEXTRA = NCHUNK - KFULL * NS
ROWS_OUT = 624
ROWS_REM = N - NS * ROWS_OUT


def _make_scatter_kernel():
    mesh = plsc.VectorSubcoreMesh(core_axis_name="c", subcore_axis_name="s")

    @functools.partial(
        pl.kernel,
        mesh=mesh,
        out_type=jax.ShapeDtypeStruct((B, N, D), jnp.float32),
        scratch_types=[
            pltpu.VMEM((CHUNK, D), jnp.float32),
            pltpu.VMEM((CHUNK, D), jnp.float32),
            pltpu.VMEM((SUBS, SUB), jnp.int32),
            pltpu.VMEM((SUBS, SUB), jnp.int32),
            pltpu.VMEM_SHARED((N, D), jnp.float32),
            pltpu.SemaphoreType.DMA,
            pltpu.SemaphoreType.DMA,
            pltpu.SemaphoreType.DMA,
            pltpu.SemaphoreType.DMA,
        ],
    )
    def scatter_kernel(msg_hbm, idx_hbm, out_hbm,
                       msg_v0, msg_v1, idx_v0, idx_v1, acc_sh,
                       sem_in0, sem_in1, sem_sc0, sem_sc1):
        c = lax.axis_index("c")
        s = lax.axis_index("s")
        bufs = ((msg_v0, idx_v0, sem_in0, sem_sc0),
                (msg_v1, idx_v1, sem_in1, sem_sc1))

        def chunk_id(k):
            return k * NS + s

        def issue_in(k, b):
            m = chunk_id(k)
            off = pl.multiple_of(m * CHUNK, 8)
            msg_v, idx_v, sem, _ = bufs[b]
            pltpu.async_copy(msg_hbm.at[c, pl.ds(off, CHUNK)], msg_v, sem)
            r0 = pl.multiple_of(m * SUBS, 8)
            pltpu.async_copy(idx_hbm.at[c, pl.ds(r0, SUBS)], idx_v, sem)

        def wait_in(b):
            msg_v, idx_v, sem, _ = bufs[b]
            pltpu.make_async_copy(msg_hbm.at[c, pl.ds(0, CHUNK)], msg_v, sem).wait()
            pltpu.make_async_copy(idx_hbm.at[c, pl.ds(0, SUBS)], idx_v, sem).wait()

        def issue_scatters(b):
            msg_v, idx_v, _, sem = bufs[b]
            return [
                pltpu.async_copy(
                    msg_v.at[pl.ds(j * SUB, SUB)],
                    acc_sh.at[idx_v.at[j]],
                    sem,
                    add=True,
                )
                for j in range(SUBS)
            ]

        def scatter_chunk(b):
            for h in issue_scatters(b):
                h.wait()

        issue_in(0, 0)
        zero16 = jnp.zeros((16,), jnp.float32)

        def zrow(r, _):
            for j in range(D // 16):
                msg_v1[r, pl.ds(j * 16, 16)] = zero16
            return _

        lax.fori_loop(0, CHUNK, zrow, None)
        zbase = pl.multiple_of(s * ROWS_OUT, 8)
        zhandles = []
        for q in range(ROWS_OUT // CHUNK):
            zhandles.append(pltpu.async_copy(
                msg_v1.at[pl.ds(0, CHUNK)],
                acc_sh.at[pl.ds(zbase + q * CHUNK, CHUNK)],
                sem_sc0,
            ))
        ztail = ROWS_OUT % CHUNK
        if ztail:
            zhandles.append(pltpu.async_copy(
                msg_v1.at[pl.ds(0, ztail)],
                acc_sh.at[pl.ds(zbase + ROWS_OUT - ztail, ztail)],
                sem_sc0,
            ))

        @pl.when(s == NS - 1)
        def _():
            pltpu.async_copy(
                msg_v1.at[pl.ds(0, ROWS_REM)],
                acc_sh.at[pl.ds(NS * ROWS_OUT, ROWS_REM)],
                sem_sc0,
            ).wait()

        for h in zhandles:
            h.wait()

        issue_in(1, 1)
        plsc.subcore_barrier()

        def pair_body(t, _):
            k0 = t * 2
            k1 = t * 2 + 1
            wait_in(0)
            h0 = issue_scatters(0)
            wait_in(1)
            h1 = issue_scatters(1)
            for h in h0:
                h.wait()
            nk0 = k0 + 2

            @pl.when((nk0 < KFULL) | ((nk0 == KFULL) & (s < NEXTRA)))
            def _():
                issue_in(nk0, 0)

            for h in h1:
                h.wait()
            nk1 = k1 + 2

            @pl.when(nk1 < KFULL)
            def _():
                issue_in(nk1, 1)

            return _

        lax.fori_loop(0, KFULL // 2, pair_body, None)

        @pl.when(s < NEXTRA)
        def _():
            wait_in(0)
            scatter_chunk(0)

        plsc.subcore_barrier()

        obase = pl.multiple_of(s * ROWS_OUT, 8)
        pltpu.sync_copy(
            acc_sh.at[pl.ds(obase, ROWS_OUT)],
            out_hbm.at[c, pl.ds(obase, ROWS_OUT)],
        )

        @pl.when(s == NS - 1)
        def _():
            pltpu.sync_copy(
                acc_sh.at[pl.ds(NS * ROWS_OUT, ROWS_REM)],
                out_hbm.at[c, pl.ds(NS * ROWS_OUT, ROWS_REM)],
            )

    return scatter_kernel


_scatter = _make_scatter_kernel()


def kernel(messages, tgt_indices, atom_features_ref):
    del atom_features_ref
    idx3 = tgt_indices.reshape(B, E // SUB, SUB)
    return _scatter(messages, idx3)


# speedup vs baseline: 1.2694x; 1.2694x over previous
import functools

import jax
import jax.numpy as jnp
from jax import lax
from jax.experimental import pallas as pl
from jax.experimental.pallas import tpu as pltpu
from jax.experimental.pallas import tpu_sc as plsc

B, E, N, D = 2, 160000, 10000, 128
NS = 16
CHUNK = 160
SUB = 20
SUBS = CHUNK // SUB
NCHUNK = E // CHUNK
KFULL = NCHUNK // NS
NEXTRA = NCHUNK - KFULL * NS
ROWS_OUT = 624
ROWS_REM = N - NS * ROWS_OUT


def _make_scatter_kernel():
    mesh = plsc.VectorSubcoreMesh(core_axis_name="c", subcore_axis_name="s")

    @functools.partial(
        pl.kernel,
        mesh=mesh,
        out_type=jax.ShapeDtypeStruct((B, N, D), jnp.float32),
        scratch_types=[
            pltpu.VMEM((CHUNK, D), jnp.float32),
            pltpu.VMEM((CHUNK, D), jnp.float32),
            pltpu.VMEM((SUBS, SUB), jnp.int32),
            pltpu.VMEM((SUBS, SUB), jnp.int32),
            pltpu.VMEM_SHARED((N, D), jnp.float32),
            pltpu.SemaphoreType.DMA,
            pltpu.SemaphoreType.DMA,
            pltpu.SemaphoreType.DMA,
        ],
    )
    def scatter_kernel(msg_hbm, idx_hbm, out_hbm,
                       msg_v0, msg_v1, idx_v0, idx_v1, acc_sh,
                       sem_in0, sem_in1, sem_sc):
        c = lax.axis_index("c")
        s = lax.axis_index("s")
        bufs = ((msg_v0, idx_v0, sem_in0), (msg_v1, idx_v1, sem_in1))

        def chunk_id(k):
            return k * NS + s

        def issue_in(k, b):
            m = chunk_id(k)
            off = pl.multiple_of(m * CHUNK, 8)
            msg_v, idx_v, sem = bufs[b]
            pltpu.async_copy(msg_hbm.at[c, pl.ds(off, CHUNK)], msg_v, sem)
            r0 = pl.multiple_of(m * SUBS, 8)
            pltpu.async_copy(idx_hbm.at[c, pl.ds(r0, SUBS)], idx_v, sem)

        def wait_in(b):
            msg_v, idx_v, sem = bufs[b]
            pltpu.make_async_copy(msg_hbm.at[c, pl.ds(0, CHUNK)], msg_v, sem).wait()
            pltpu.make_async_copy(idx_hbm.at[c, pl.ds(0, SUBS)], idx_v, sem).wait()

        def scatter_chunk(b):
            msg_v, idx_v, _ = bufs[b]
            handles = [
                pltpu.async_copy(
                    msg_v.at[pl.ds(j * SUB, SUB)],
                    acc_sh.at[idx_v.at[j]],
                    sem_sc,
                    add=True,
                )
                for j in range(SUBS)
            ]
            for h in handles:
                h.wait()

        issue_in(0, 0)
        zero16 = jnp.zeros((16,), jnp.float32)

        def zrow(r, _):
            for j in range(D // 16):
                msg_v1[r, pl.ds(j * 16, 16)] = zero16
            return _

        lax.fori_loop(0, CHUNK, zrow, None)
        zbase = pl.multiple_of(s * ROWS_OUT, 8)
        zhandles = []
        for q in range(ROWS_OUT // CHUNK):
            zhandles.append(pltpu.async_copy(
                msg_v1.at[pl.ds(0, CHUNK)],
                acc_sh.at[pl.ds(zbase + q * CHUNK, CHUNK)],
                sem_sc,
            ))
        ztail = ROWS_OUT % CHUNK
        if ztail:
            zhandles.append(pltpu.async_copy(
                msg_v1.at[pl.ds(0, ztail)],
                acc_sh.at[pl.ds(zbase + ROWS_OUT - ztail, ztail)],
                sem_sc,
            ))

        @pl.when(s == NS - 1)
        def _():
            pltpu.async_copy(
                msg_v1.at[pl.ds(0, ROWS_REM)],
                acc_sh.at[pl.ds(NS * ROWS_OUT, ROWS_REM)],
                sem_sc,
            ).wait()

        for h in zhandles:
            h.wait()

        issue_in(1, 1)
        plsc.subcore_barrier()

        def process(k, b, next_k):
            wait_in(b)
            scatter_chunk(b)
            if next_k is not None:
                if isinstance(next_k, tuple):
                    nk, pred = next_k

                    @pl.when(pred)
                    def _():
                        issue_in(nk, b)
                else:
                    issue_in(next_k, b)

        def pair_body(t, _):
            k0 = t * 2
            k1 = t * 2 + 1
            nk0 = k0 + 2
            process(k0, 0, (nk0, (nk0 < KFULL) | ((nk0 == KFULL) & (s < NEXTRA))))
            nk1 = k1 + 2
            process(k1, 1, (nk1, nk1 < KFULL))
            return _

        lax.fori_loop(0, KFULL // 2, pair_body, None)

        @pl.when(s < NEXTRA)
        def _():
            process(KFULL, 0, None)

        plsc.subcore_barrier()

        obase = pl.multiple_of(s * ROWS_OUT, 8)
        pltpu.sync_copy(
            acc_sh.at[pl.ds(obase, ROWS_OUT)],
            out_hbm.at[c, pl.ds(obase, ROWS_OUT)],
        )

        @pl.when(s == NS - 1)
        def _():
            pltpu.sync_copy(
                acc_sh.at[pl.ds(NS * ROWS_OUT, ROWS_REM)],
                out_hbm.at[c, pl.ds(NS * ROWS_OUT, ROWS_REM)],
            )

    return scatter_kernel


_scatter = _make_scatter_kernel()


def kernel(messages, tgt_indices, atom_features_ref):
    del atom_features_ref
    idx3 = tgt_indices.reshape(B, E // SUB, SUB)
    return _scatter(messages, idx3)
